# baseline (speedup 1.0000x reference)
import functools

import jax
import jax.numpy as jnp
from jax import lax
from jax.experimental import pallas as pl
from jax.experimental.pallas import tpu as pltpu
from jax.experimental.pallas import tpu_sc as plsc

_NC = 2
_NS = 16
_NW = _NC * _NS
_DP = 128


def _gather_t(x_t, table, N, J, D):
    nblk = N // _DP // _NW
    mesh = plsc.VectorSubcoreMesh(
        core_axis_name="c", subcore_axis_name="s",
        num_cores=_NC, num_subcores=_NS)

    @functools.partial(
        pl.kernel,
        out_type=jax.ShapeDtypeStruct((J, D, N), jnp.float32),
        mesh=mesh,
        scratch_types=[
            pltpu.VMEM((J, _DP), jnp.int32),
            pltpu.VMEM((2, _DP, _DP), jnp.float32),
            pltpu.VMEM((2, D, _DP), jnp.float32),
            [pltpu.SemaphoreType.DMA] * 2,
            [pltpu.SemaphoreType.DMA] * 2,
        ],
        compiler_params=pltpu.CompilerParams(
            use_tc_tiling_on_sc=True, needs_layout_passes=False),
    )
    def k(x_hbm, w_hbm, out_hbm, idx_v, rows_v, tiles_v, semg, sems):
        wid = lax.axis_index("s") * _NC + lax.axis_index("c")
        iota = lax.iota(jnp.int32, 16)
        rowv = [iota + (icg * 16) for icg in range(8)]

        def fire_g(u, b):
            pltpu.async_copy(w_hbm.at[idx_v.at[u]], rows_v.at[b], semg[b])

        def wait_g(b):
            pltpu.make_async_copy(
                w_hbm.at[idx_v.at[0]], rows_v.at[b], semg[b]).wait()

        def fire_s(u, b, i0):
            pltpu.async_copy(
                tiles_v.at[b], out_hbm.at[u, :, pl.ds(i0, _DP)], sems[b])

        def wait_s(b, i0):
            pltpu.make_async_copy(
                tiles_v.at[b], out_hbm.at[0, :, pl.ds(i0, _DP)],
                sems[b]).wait()

        def transpose(b):
            @pl.loop(0, D, unroll=4)
            def _col(kk):
                colv = jnp.full((16,), 0, jnp.int32) + kk
                for icg in range(8):
                    v = plsc.load_gather(rows_v.at[b], [rowv[icg], colv])
                    tiles_v[b, kk, pl.ds(icg * 16, 16)] = v

        for blk in range(nblk):
            i0 = (wid * nblk + blk) * _DP
            pltpu.sync_copy(x_hbm.at[:, pl.ds(i0, _DP)], idx_v)
            fire_g(0, 0)
            fire_g(1, 1)
            for u in range(2):
                wait_g(u % 2)
                transpose(u % 2)
                fire_s(u, u % 2, i0)
                fire_g(u + 2, u % 2)

            @pl.loop(2, J - 2)
            def _unit(u):
                for b in range(2):
                    @pl.when((u % 2) == b)
                    def _():
                        wait_g(b)
                        wait_s(b, i0)
                        transpose(b)
                        fire_s(u, b, i0)
                        fire_g(u + 2, b)

            for uu in range(J - 2, J):
                b = uu % 2
                wait_g(b)
                wait_s(b, i0)
                transpose(b)
                fire_s(uu, b, i0)
            for b in range(2):
                wait_s(b, i0)

    return k(x_t, table)


def kernel(x, W):
    N, J = x.shape
    D = W.shape[1]
    x_t = x.T.astype(jnp.int32)
    W_pad = jnp.pad(W, ((0, 0), (0, _DP - D)))
    P = _gather_t(x_t, W_pad, N, J, D)
    return P.transpose(2, 0, 1)


# speedup vs baseline: 1.8719x; 1.8719x over previous
import functools

import jax
import jax.numpy as jnp
from jax import lax
from jax.experimental import pallas as pl
from jax.experimental.pallas import tpu as pltpu
from jax.experimental.pallas import tpu_sc as plsc

_NC = 2
_NS = 16
_NW = _NC * _NS
_DP = 128


def _gather_t(x_t, table, N, J, D):
    nblk = N // _DP // _NW
    mesh = plsc.VectorSubcoreMesh(
        core_axis_name="c", subcore_axis_name="s",
        num_cores=_NC, num_subcores=_NS)

    @functools.partial(
        pl.kernel,
        out_type=jax.ShapeDtypeStruct((J, D, N), jnp.float32),
        mesh=mesh,
        scratch_types=[
            pltpu.VMEM((J, _DP), jnp.int32),
            pltpu.VMEM((2, _DP, _DP), jnp.float32),
            pltpu.VMEM((2, D, _DP), jnp.float32),
            [pltpu.SemaphoreType.DMA] * 2,
            [pltpu.SemaphoreType.DMA] * 2,
        ],
        compiler_params=pltpu.CompilerParams(
            use_tc_tiling_on_sc=True, needs_layout_passes=False),
    )
    def k(x_hbm, w_hbm, out_hbm, idx_v, rows_v, tiles_v, semg, sems):
        wid = lax.axis_index("s") * _NC + lax.axis_index("c")
        iota = lax.iota(jnp.int32, 16)
        rowv = [iota + (icg * 16) for icg in range(8)]

        def fire_g(u, b):
            pltpu.async_copy(w_hbm.at[idx_v.at[u]], rows_v.at[b], semg[b])

        def wait_g(b):
            pltpu.make_async_copy(
                w_hbm.at[idx_v.at[0]], rows_v.at[b], semg[b]).wait()

        def fire_s(u, b, i0):
            pltpu.async_copy(
                tiles_v.at[b], out_hbm.at[u, :, pl.ds(i0, _DP)], sems[b])

        def wait_s(b, i0):
            pltpu.make_async_copy(
                tiles_v.at[b], out_hbm.at[0, :, pl.ds(i0, _DP)],
                sems[b]).wait()

        def transpose(b):
            @plsc.parallel_loop(0, D, unroll=4)
            def _col(kk):
                colv = jnp.full((16,), 0, jnp.int32) + kk
                for icg in range(8):
                    v = plsc.load_gather(rows_v.at[b], [rowv[icg], colv])
                    tiles_v[b, kk, pl.ds(icg * 16, 16)] = v

        for blk in range(nblk):
            i0 = (wid * nblk + blk) * _DP
            pltpu.sync_copy(x_hbm.at[:, pl.ds(i0, _DP)], idx_v)
            fire_g(0, 0)
            fire_g(1, 1)
            for u in range(2):
                wait_g(u % 2)
                transpose(u % 2)
                fire_s(u, u % 2, i0)
                fire_g(u + 2, u % 2)

            @pl.loop(2, J - 2)
            def _unit(u):
                for b in range(2):
                    @pl.when((u % 2) == b)
                    def _():
                        wait_g(b)
                        wait_s(b, i0)
                        transpose(b)
                        fire_s(u, b, i0)
                        fire_g(u + 2, b)

            for uu in range(J - 2, J):
                b = uu % 2
                wait_g(b)
                wait_s(b, i0)
                transpose(b)
                fire_s(uu, b, i0)
            for b in range(2):
                wait_s(b, i0)

    return k(x_t, table)


def kernel(x, W):
    N, J = x.shape
    D = W.shape[1]
    x_t = x.T.astype(jnp.int32)
    W_pad = jnp.pad(W, ((0, 0), (0, _DP - D)))
    P = _gather_t(x_t, W_pad, N, J, D)
    return P.transpose(2, 0, 1)
